# baseline (speedup 1.0000x reference)
import functools

import jax
import jax.numpy as jnp
from jax.experimental import pallas as pl

_E = 8
_K = 2
_H = 64


def _moe_kernel(x_ref, wg_ref, w1_ref, w2_ref, out_ref, ep_ref, loss_ref,
                *, nblk, ntok):
    i = pl.program_id(0)
    xb = x_ref[...]

    logits = jnp.dot(xb, wg_ref[...], preferred_element_type=jnp.float32)
    gate = jax.nn.softmax(logits, axis=-1)

    lane = jax.lax.broadcasted_iota(jnp.int32, gate.shape, 1)
    big = jnp.int32(_E)
    m1 = jnp.max(gate, axis=1, keepdims=True)
    i1 = jnp.min(jnp.where(gate == m1, lane, big), axis=1, keepdims=True)
    sel1 = lane == i1
    gate2 = jnp.where(sel1, -jnp.inf, gate)
    m2 = jnp.max(gate2, axis=1, keepdims=True)
    i2 = jnp.min(jnp.where(gate2 == m2, lane, big), axis=1, keepdims=True)
    sel2 = lane == i2
    cw = jnp.where(sel1, m1, 0.0) + jnp.where(sel2, m2, 0.0)

    @pl.when(i == 0)
    def _():
        ep_ref[...] = jnp.zeros_like(ep_ref)

    ep_ref[...] += jnp.sum(gate, axis=0, keepdims=True)

    rep = (jax.lax.broadcasted_iota(jnp.int32, (_E, _E * _H), 1) // _H
           == jax.lax.broadcasted_iota(jnp.int32, (_E, _E * _H), 0)
           ).astype(jnp.float32)
    scale = jnp.dot(cw, rep, preferred_element_type=jnp.float32)

    h = jnp.dot(xb, w1_ref[...], preferred_element_type=jnp.float32)
    h = h * jax.nn.sigmoid(h)
    out_ref[...] = jnp.dot(h * scale, w2_ref[...],
                           preferred_element_type=jnp.float32)

    @pl.when(i == nblk - 1)
    def _():
        ep = ep_ref[...] / ntok
        m = jnp.mean(ep)
        var = jnp.mean((ep - m) ** 2)
        loss_ref[...] = jnp.full_like(loss_ref, var / (m * m + 1e-10))


def kernel(x, Wg, W1, W2):
    B, S, D = x.shape
    N = B * S
    T = 2048
    nblk = N // T

    xf = x.reshape(N, D)
    wg_t = Wg.T
    w1_t = W1.reshape(_E * _H, D).T
    w2_r = jnp.transpose(W2, (0, 2, 1)).reshape(_E * _H, D)

    out, _, loss = pl.pallas_call(
        functools.partial(_moe_kernel, nblk=nblk, ntok=N),
        grid=(nblk,),
        in_specs=[
            pl.BlockSpec((T, D), lambda i: (i, 0)),
            pl.BlockSpec((D, _E), lambda i: (0, 0)),
            pl.BlockSpec((D, _E * _H), lambda i: (0, 0)),
            pl.BlockSpec((_E * _H, D), lambda i: (0, 0)),
        ],
        out_specs=[
            pl.BlockSpec((T, D), lambda i: (i, 0)),
            pl.BlockSpec((1, _E), lambda i: (0, 0)),
            pl.BlockSpec((1, 1), lambda i: (0, 0)),
        ],
        out_shape=[
            jax.ShapeDtypeStruct((N, D), jnp.float32),
            jax.ShapeDtypeStruct((1, _E), jnp.float32),
            jax.ShapeDtypeStruct((1, 1), jnp.float32),
        ],
    )(xf, wg_t, w1_t, w2_r)

    return out.reshape(B, S, D), loss[0, 0]


# speedup vs baseline: 1.1353x; 1.1353x over previous
import functools

import jax
import jax.numpy as jnp
from jax.experimental import pallas as pl

_E = 8
_K = 2
_H = 64
_EH = _E * _H
_PADE = 128


def _moe_kernel(x_ref, wcat_ref, w2_ref, out_ref, ep_ref, loss_ref,
                *, nblk, ntok, tblk):
    i = pl.program_id(0)
    xb = x_ref[...]

    hcat = jnp.dot(xb, wcat_ref[...], preferred_element_type=jnp.float32)
    hpre = hcat[:, :_EH]
    logits = hcat[:, _EH:_EH + _E]

    lane = jax.lax.broadcasted_iota(jnp.int32, logits.shape, 1)
    big = jnp.int32(_E)
    lmax = jnp.max(logits, axis=1, keepdims=True)
    el = jnp.exp(logits - lmax)
    ones_e = jnp.ones((_E, _E), jnp.float32)
    s = jnp.dot(el, ones_e, preferred_element_type=jnp.float32)
    rinv = 1.0 / s
    i1 = jnp.min(jnp.where(logits == lmax, lane, big), axis=1, keepdims=True)
    sel1 = lane == i1
    l2 = jnp.where(sel1, -jnp.inf, logits)
    lmax2 = jnp.max(l2, axis=1, keepdims=True)
    i2 = jnp.min(jnp.where(l2 == lmax2, lane, big), axis=1, keepdims=True)
    sel2 = lane == i2
    m2 = jnp.exp(lmax2 - lmax) * rinv
    cw = jnp.where(sel1, rinv, 0.0) + jnp.where(sel2, m2, 0.0)

    gate = el * rinv
    ones_t = jnp.ones((1, tblk), jnp.float32)
    ep_blk = jnp.dot(ones_t, gate, preferred_element_type=jnp.float32)

    @pl.when(i == 0)
    def _():
        ep_ref[...] = jnp.zeros_like(ep_ref)

    ep_ref[...] += ep_blk

    rep = (jax.lax.broadcasted_iota(jnp.int32, (_E, _EH), 1) // _H
           == jax.lax.broadcasted_iota(jnp.int32, (_E, _EH), 0)
           ).astype(jnp.float32)
    scale = jnp.dot(cw, rep, preferred_element_type=jnp.float32)

    h = hpre * jax.nn.sigmoid(hpre)
    out_ref[...] = jnp.dot(h * scale, w2_ref[...],
                           preferred_element_type=jnp.float32)

    @pl.when(i == nblk - 1)
    def _():
        ep = ep_ref[...] / ntok
        m = jnp.mean(ep)
        var = jnp.mean((ep - m) ** 2)
        loss_ref[...] = jnp.full_like(loss_ref, var / (m * m + 1e-10))


def kernel(x, Wg, W1, W2):
    B, S, D = x.shape
    N = B * S
    T = 1024
    nblk = N // T

    xf = x.reshape(N, D)
    wg_pad = jnp.zeros((D, _PADE), x.dtype).at[:, :_E].set(Wg.T)
    wcat = jnp.concatenate([W1.reshape(_EH, D).T, wg_pad], axis=1)
    w2_r = jnp.transpose(W2, (0, 2, 1)).reshape(_EH, D)

    out, _, loss = pl.pallas_call(
        functools.partial(_moe_kernel, nblk=nblk, ntok=N, tblk=T),
        grid=(nblk,),
        in_specs=[
            pl.BlockSpec((T, D), lambda i: (i, 0)),
            pl.BlockSpec((D, _EH + _PADE), lambda i: (0, 0)),
            pl.BlockSpec((_EH, D), lambda i: (0, 0)),
        ],
        out_specs=[
            pl.BlockSpec((T, D), lambda i: (i, 0)),
            pl.BlockSpec((1, _E), lambda i: (0, 0)),
            pl.BlockSpec((1, 1), lambda i: (0, 0)),
        ],
        out_shape=[
            jax.ShapeDtypeStruct((N, D), jnp.float32),
            jax.ShapeDtypeStruct((1, _E), jnp.float32),
            jax.ShapeDtypeStruct((1, 1), jnp.float32),
        ],
    )(xf, wcat, w2_r)

    return out.reshape(B, S, D), loss[0, 0]


# speedup vs baseline: 1.1385x; 1.0028x over previous
import functools

import jax
import jax.numpy as jnp
from jax.experimental import pallas as pl

_E = 8
_K = 2
_H = 64
_EH = _E * _H
_PADE = 128


def _moe_kernel(x_ref, wcat_ref, w2_ref, out_ref, ep_ref, loss_ref,
                *, nblk, ntok, tblk):
    i = pl.program_id(0)
    xb = x_ref[...]

    hcat = jnp.dot(xb, wcat_ref[...], preferred_element_type=jnp.float32)
    hpre = hcat[:, :_EH]
    logits = hcat[:, _EH:_EH + _E]

    lt = jnp.transpose(logits, (1, 0))
    sub = jax.lax.broadcasted_iota(jnp.int32, lt.shape, 0)
    big = jnp.int32(_E)
    lmax = jnp.max(lt, axis=0, keepdims=True)
    el = jnp.exp(lt - lmax)
    s = jnp.sum(el, axis=0, keepdims=True)
    rinv = 1.0 / s
    i1 = jnp.min(jnp.where(lt == lmax, sub, big), axis=0, keepdims=True)
    sel1 = sub == i1
    l2 = jnp.where(sel1, -jnp.inf, lt)
    lmax2 = jnp.max(l2, axis=0, keepdims=True)
    i2 = jnp.min(jnp.where(l2 == lmax2, sub, big), axis=0, keepdims=True)
    sel2 = sub == i2
    m2 = jnp.exp(lmax2 - lmax) * rinv
    cwt = jnp.where(sel1, rinv, 0.0) + jnp.where(sel2, m2, 0.0)

    gate_t = el * rinv
    ones_t = jnp.ones((tblk, 1), jnp.float32)
    ep_blk = jnp.dot(gate_t, ones_t, preferred_element_type=jnp.float32)

    @pl.when(i == 0)
    def _():
        ep_ref[...] = jnp.zeros_like(ep_ref)

    ep_ref[...] += ep_blk

    rep = (jax.lax.broadcasted_iota(jnp.int32, (_E, _EH), 1) // _H
           == jax.lax.broadcasted_iota(jnp.int32, (_E, _EH), 0)
           ).astype(jnp.float32)
    scale = jax.lax.dot_general(
        cwt, rep, (((0,), (0,)), ((), ())),
        preferred_element_type=jnp.float32)

    h = hpre * jax.nn.sigmoid(hpre)
    out_ref[...] = jnp.dot(h * scale, w2_ref[...],
                           preferred_element_type=jnp.float32)

    @pl.when(i == nblk - 1)
    def _():
        ep = ep_ref[...] / ntok
        m = jnp.mean(ep)
        var = jnp.mean((ep - m) ** 2)
        loss_ref[...] = jnp.full_like(loss_ref, var / (m * m + 1e-10))


def kernel(x, Wg, W1, W2):
    B, S, D = x.shape
    N = B * S
    T = 1024
    nblk = N // T

    xf = x.reshape(N, D)
    wg_pad = jnp.zeros((D, _PADE), x.dtype).at[:, :_E].set(Wg.T)
    wcat = jnp.concatenate([W1.reshape(_EH, D).T, wg_pad], axis=1)
    w2_r = jnp.transpose(W2, (0, 2, 1)).reshape(_EH, D)

    out, _, loss = pl.pallas_call(
        functools.partial(_moe_kernel, nblk=nblk, ntok=N, tblk=T),
        grid=(nblk,),
        in_specs=[
            pl.BlockSpec((T, D), lambda i: (i, 0)),
            pl.BlockSpec((D, _EH + _PADE), lambda i: (0, 0)),
            pl.BlockSpec((_EH, D), lambda i: (0, 0)),
        ],
        out_specs=[
            pl.BlockSpec((T, D), lambda i: (i, 0)),
            pl.BlockSpec((_E, 1), lambda i: (0, 0)),
            pl.BlockSpec((1, 1), lambda i: (0, 0)),
        ],
        out_shape=[
            jax.ShapeDtypeStruct((N, D), jnp.float32),
            jax.ShapeDtypeStruct((_E, 1), jnp.float32),
            jax.ShapeDtypeStruct((1, 1), jnp.float32),
        ],
    )(xf, wcat, w2_r)

    return out.reshape(B, S, D), loss[0, 0]
